# baseline (speedup 1.0000x reference)
import jax
import jax.numpy as jnp
from jax.experimental import pallas as pl
from jax.experimental.pallas import tpu as pltpu

BATCH = 320
P = 32
D = 128
E = 128
BBLK = 16


def _gin_chunk(h2, we, be, wn, bn, nb):
    A = jnp.dot(h2, we[:D, :], preferred_element_type=jnp.float32)
    S = jnp.dot(h2, we[D:, :], preferred_element_type=jnp.float32)
    T = (A + be).reshape(nb, P, E)
    S3 = S.reshape(nb, P, E)

    Tb = T.astype(jnp.bfloat16)
    Sb = S3.astype(jnp.bfloat16)
    terms = [jax.nn.relu(Tb + Sb[:, i:i + 1, :]) for i in range(P)]
    while len(terms) > 1:
        terms = [terms[k] + terms[k + 1] for k in range(0, len(terms), 2)]
    agg = terms[0].astype(jnp.float32) - jax.nn.relu(T + S3)

    agg2 = agg.reshape(nb * P, E)
    out = (
        jnp.dot(h2, wn[:D, :], preferred_element_type=jnp.float32)
        + jnp.dot(agg2, wn[D:, :], preferred_element_type=jnp.float32)
        + bn
    )
    return jax.nn.relu(out).reshape(nb, P, D)


def _gin_block_kernel(h_ref, we_ref, be_ref, wn_ref, bn_ref, out_ref):
    HB = 4
    we = we_ref[...]
    be = be_ref[...]
    wn = wn_ref[...]
    bn = bn_ref[...]
    for q in range(BBLK // HB):
        h2q = h_ref[q * HB:(q + 1) * HB].reshape(HB * P, D)
        out_ref[q * HB:(q + 1) * HB] = _gin_chunk(h2q, we, be, wn, bn, HB)


def kernel(t, h, W_edge, b_edge, W_node, b_node):
    del t
    be2 = b_edge.reshape(1, E)
    bn2 = b_node.reshape(1, D)
    return pl.pallas_call(
        _gin_block_kernel,
        out_shape=jax.ShapeDtypeStruct((BATCH, P, D), jnp.float32),
        grid=(BATCH // BBLK,),
        in_specs=[
            pl.BlockSpec((BBLK, P, D), lambda i: (i, 0, 0)),
            pl.BlockSpec((2 * D, E), lambda i: (0, 0)),
            pl.BlockSpec((1, E), lambda i: (0, 0)),
            pl.BlockSpec((D + E, D), lambda i: (0, 0)),
            pl.BlockSpec((1, D), lambda i: (0, 0)),
        ],
        out_specs=pl.BlockSpec((BBLK, P, D), lambda i: (i, 0, 0)),
        compiler_params=pltpu.CompilerParams(
            dimension_semantics=("parallel",),
        ),
    )(h, W_edge, be2, W_node, bn2)


# speedup vs baseline: 1.3478x; 1.3478x over previous
import jax
import jax.numpy as jnp
from jax.experimental import pallas as pl
from jax.experimental.pallas import tpu as pltpu

BATCH = 320
P = 32
D = 128
E = 128
BBLK = 64


def _gin_chunk(h2, we, be, wn, bn, nb):
    A = jnp.dot(h2, we[:D, :], preferred_element_type=jnp.float32)
    S = jnp.dot(h2, we[D:, :], preferred_element_type=jnp.float32)

    Tb = A.astype(jnp.bfloat16).reshape(nb, P, E) + be.astype(jnp.bfloat16)
    Sb = S.astype(jnp.bfloat16).reshape(nb, P, E)
    terms = [-jax.nn.relu(Tb + Sb)]
    terms += [jax.nn.relu(Tb + Sb[:, i:i + 1, :]) for i in range(P)]
    while len(terms) > 1:
        terms = [terms[k] + terms[k + 1] for k in range(0, len(terms) - 1, 2)] + (
            [terms[-1]] if len(terms) % 2 else []
        )
    agg = terms[0].astype(jnp.float32)

    agg2 = agg.reshape(nb * P, E)
    out = (
        jnp.dot(h2, wn[:D, :], preferred_element_type=jnp.float32)
        + jnp.dot(agg2, wn[D:, :], preferred_element_type=jnp.float32)
        + bn
    )
    return jax.nn.relu(out).reshape(nb, P, D)


def _gin_block_kernel(h_ref, we_ref, be_ref, wn_ref, bn_ref, out_ref):
    HB = 4
    we = we_ref[...]
    be = be_ref[...]
    wn = wn_ref[...]
    bn = bn_ref[...]
    for q in range(BBLK // HB):
        h2q = h_ref[q * HB:(q + 1) * HB].reshape(HB * P, D)
        out_ref[q * HB:(q + 1) * HB] = _gin_chunk(h2q, we, be, wn, bn, HB)


def kernel(t, h, W_edge, b_edge, W_node, b_node):
    del t
    be2 = b_edge.reshape(1, E)
    bn2 = b_node.reshape(1, D)
    return pl.pallas_call(
        _gin_block_kernel,
        out_shape=jax.ShapeDtypeStruct((BATCH, P, D), jnp.float32),
        grid=(BATCH // BBLK,),
        in_specs=[
            pl.BlockSpec((BBLK, P, D), lambda i: (i, 0, 0)),
            pl.BlockSpec((2 * D, E), lambda i: (0, 0)),
            pl.BlockSpec((1, E), lambda i: (0, 0)),
            pl.BlockSpec((D + E, D), lambda i: (0, 0)),
            pl.BlockSpec((1, D), lambda i: (0, 0)),
        ],
        out_specs=pl.BlockSpec((BBLK, P, D), lambda i: (i, 0, 0)),
        compiler_params=pltpu.CompilerParams(
            dimension_semantics=("parallel",),
        ),
    )(h, W_edge, be2, W_node, bn2)
